# baseline (speedup 1.0000x reference)
import functools

import jax
import jax.numpy as jnp
from jax.experimental import pallas as pl
from jax.experimental.pallas import tpu as pltpu


def _mm_bias_kernel(x_ref, w_ref, b_ref, o_ref):
    o_ref[...] = (
        jnp.dot(x_ref[...], w_ref[...], preferred_element_type=jnp.float32)
        + b_ref[...]
    )


def _matmul_bias(x, w_t, b, bm=400):
    M, K = x.shape
    N = w_t.shape[1]
    return pl.pallas_call(
        _mm_bias_kernel,
        grid=(M // bm,),
        in_specs=[
            pl.BlockSpec((bm, K), lambda i: (i, 0)),
            pl.BlockSpec((K, N), lambda i: (0, 0)),
            pl.BlockSpec((1, N), lambda i: (0, 0)),
        ],
        out_specs=pl.BlockSpec((bm, N), lambda i: (i, 0)),
        out_shape=jax.ShapeDtypeStruct((M, N), jnp.float32),
    )(x, w_t, b[None, :])


def _post_kernel(s_ref, w_ref, x_ref, cnt_ref, b_ref, o_ref):
    o_ref[...] = (
        x_ref[...]
        + jnp.dot(s_ref[...], w_ref[...], preferred_element_type=jnp.float32)
        + cnt_ref[...] * b_ref[...]
    )


def _post_matmul(s, w2_t, x, cnt, b2, bm=400):
    M, K = s.shape
    N = w2_t.shape[1]
    return pl.pallas_call(
        _post_kernel,
        grid=(M // bm,),
        in_specs=[
            pl.BlockSpec((bm, K), lambda i: (i, 0)),
            pl.BlockSpec((K, N), lambda i: (0, 0)),
            pl.BlockSpec((bm, N), lambda i: (i, 0)),
            pl.BlockSpec((bm, 1), lambda i: (i, 0)),
            pl.BlockSpec((1, N), lambda i: (0, 0)),
        ],
        out_specs=pl.BlockSpec((bm, N), lambda i: (i, 0)),
        out_shape=jax.ShapeDtypeStruct((M, N), jnp.float32),
    )(s, w2_t, x, cnt[:, None], b2[None, :])


def _proj_kernel(a_ref, b_ref, wa_ref, wb_ref, bias_ref, o_ref):
    o_ref[...] = (
        jnp.dot(a_ref[...], wa_ref[...], preferred_element_type=jnp.float32)
        + jnp.dot(b_ref[...], wb_ref[...], preferred_element_type=jnp.float32)
        + bias_ref[...]
    )


def _final_proj(x_bu, x_td, wa_t, wb_t, bias, bm=400):
    M, K = x_bu.shape
    N = wa_t.shape[1]
    return pl.pallas_call(
        _proj_kernel,
        grid=(M // bm,),
        in_specs=[
            pl.BlockSpec((bm, K), lambda i: (i, 0)),
            pl.BlockSpec((bm, K), lambda i: (i, 0)),
            pl.BlockSpec((K, N), lambda i: (0, 0)),
            pl.BlockSpec((K, N), lambda i: (0, 0)),
            pl.BlockSpec((1, N), lambda i: (0, 0)),
        ],
        out_specs=pl.BlockSpec((bm, N), lambda i: (i, 0)),
        out_shape=jax.ShapeDtypeStruct((M, N), jnp.float32),
    )(x_bu, x_td, wa_t, wb_t, bias[None, :])


def _conv_round(x, a_b, src, dst, bl, w1c, w2_t, b2, cnt):
    d = x.shape[1]
    h = jnp.maximum(
        a_b[src, :d] + a_b[dst, d:] + bl[:, None] * w1c[None, :], 0.0
    )
    s = jnp.zeros_like(x).at[dst].add(h)
    return _post_matmul(s, w2_t, x, cnt, b2)


def kernel(edge_index, species_ids, branch_lengths, params):
    n = species_ids.shape[0]
    d = params["internal_embedding"].shape[0]

    x0 = jnp.take(params["species_embedding"], species_ids, axis=0)

    p2c = edge_index[:, 0::2]
    c2p = edge_index[:, 1::2]
    bl_p2c = branch_lengths[0::2]
    bl_c2p = branch_lengths[1::2]

    ones = jnp.ones((p2c.shape[1],), jnp.float32)
    cnt_bu = jnp.zeros((n,), jnp.float32).at[c2p[1]].add(ones)
    cnt_td = jnp.zeros((n,), jnp.float32).at[p2c[1]].add(ones)

    def run_dir(x, layers, ei, bl, cnt):
        src, dst = ei[0], ei[1]
        for lp in layers:
            w1ab_t = jnp.concatenate(
                [lp["W1"][:, :d].T, lp["W1"][:, d : 2 * d].T], axis=1
            )
            bias_ab = jnp.concatenate([lp["b1"], jnp.zeros((d,), jnp.float32)])
            a_b = _matmul_bias(x, w1ab_t, bias_ab)
            x = _conv_round(
                x, a_b, src, dst, bl, lp["W1"][:, 2 * d], lp["W2"].T, lp["b2"], cnt
            )
        return x

    x_bu = run_dir(x0, params["bu"], c2p, bl_c2p, cnt_bu)
    x_td = run_dir(x0, params["td"], p2c, bl_p2c, cnt_td)

    pw = params["proj_W"]
    return _final_proj(x_bu, x_td, pw[:, :d].T, pw[:, d:].T, params["proj_b"])
